# speedup vs baseline: 2.9674x; 2.9674x over previous
import functools

import jax
import jax.numpy as jnp
from jax import lax
from jax.experimental import pallas as pl
from jax.experimental.pallas import tpu as pltpu
from jax.experimental.pallas import tpu_sc as plsc

B, L, D = 4096, 50, 128
TOTAL = B * L
NC, NS = 2, 16
NW = NC * NS
PER_W = TOTAL // NW
CHUNK = 128
NCHUNK = PER_W // CHUNK


@functools.partial(
    pl.kernel,
    mesh=plsc.VectorSubcoreMesh(core_axis_name="c", subcore_axis_name="s"),
    out_type=jax.ShapeDtypeStruct((TOTAL, D), jnp.float32),
    scratch_types=[
        pltpu.VMEM((NCHUNK, CHUNK), jnp.int32),
        pltpu.VMEM((CHUNK, D), jnp.float32),
        pltpu.SemaphoreType.DMA,
    ],
)
def _gather_kernel(idx_hbm, table_hbm, out_hbm, idx_v, rows_v, sem):
    wid = lax.axis_index("s") * NC + lax.axis_index("c")
    base = wid * PER_W
    pltpu.sync_copy(idx_hbm.at[wid], idx_v)

    def body(j, carry):
        pltpu.async_copy(table_hbm.at[idx_v.at[j]], rows_v, sem).wait()
        pltpu.sync_copy(rows_v, out_hbm.at[pl.ds(base + j * CHUNK, CHUNK)])
        return carry

    lax.fori_loop(0, NCHUNK, body, 0)


def kernel(token_ids, weight):
    idx = token_ids.astype(jnp.int32).reshape(NW, NCHUNK, CHUNK)
    out = _gather_kernel(idx, weight)
    return out.reshape(B, L, D)


# speedup vs baseline: 3.3337x; 1.1234x over previous
import functools

import jax
import jax.numpy as jnp
from jax import lax
from jax.experimental import pallas as pl
from jax.experimental.pallas import tpu as pltpu
from jax.experimental.pallas import tpu_sc as plsc

B, L, D = 4096, 50, 128
TOTAL = B * L
NC, NS = 2, 16
NW = NC * NS
PER_W = TOTAL // NW
CHUNK = 64
NCHUNK = PER_W // CHUNK
NBUF = 5
NGROUP = NCHUNK // NBUF


@functools.partial(
    pl.kernel,
    mesh=plsc.VectorSubcoreMesh(core_axis_name="c", subcore_axis_name="s"),
    out_type=jax.ShapeDtypeStruct((TOTAL, D), jnp.float32),
    scratch_types=[
        pltpu.VMEM((NCHUNK, CHUNK), jnp.int32),
        pltpu.VMEM((2 * NBUF, CHUNK, D), jnp.float32),
    ]
    + [pltpu.SemaphoreType.DMA] * (2 * NBUF),
)
def _gather_kernel(idx_hbm, table_hbm, out_hbm, idx_v, bufs, *sems):
    gsems = sems[:NBUF]
    ssems = sems[NBUF:]
    wid = lax.axis_index("s") * NC + lax.axis_index("c")
    base = wid * PER_W
    pltpu.sync_copy(idx_hbm.at[wid], idx_v)

    for b in range(NBUF):
        pltpu.async_copy(table_hbm.at[idx_v.at[b]], bufs.at[b], gsems[b])

    def body(g, carry):
        p = lax.rem(g, 2)
        pn = 1 - p
        for b in range(NBUF):
            j = g * NBUF + b
            cur = p * NBUF + b
            nxt = pn * NBUF + b
            pltpu.make_async_copy(
                table_hbm.at[pl.ds(0, CHUNK)], bufs.at[cur], gsems[b]
            ).wait()

            @pl.when(g > 0)
            def _drain():
                pltpu.make_async_copy(
                    table_hbm.at[pl.ds(0, CHUNK)], bufs.at[nxt], ssems[b]
                ).wait()

            pltpu.async_copy(
                bufs.at[cur], out_hbm.at[pl.ds(base + j * CHUNK, CHUNK)], ssems[b]
            )

            @pl.when(g < NGROUP - 1)
            def _next_gather():
                pltpu.async_copy(
                    table_hbm.at[idx_v.at[j + NBUF]], bufs.at[nxt], gsems[b]
                )

        return carry

    lax.fori_loop(0, NGROUP, body, 0)

    for b in range(NBUF):
        pltpu.make_async_copy(
            table_hbm.at[pl.ds(0, CHUNK)], bufs.at[b], ssems[b]
        ).wait()


def kernel(token_ids, weight):
    idx = token_ids.astype(jnp.int32).reshape(NW, NCHUNK, CHUNK)
    out = _gather_kernel(idx, weight)
    return out.reshape(B, L, D)


# speedup vs baseline: 6.0030x; 1.8007x over previous
import functools

import jax
import jax.numpy as jnp
from jax import lax
from jax.experimental import pallas as pl
from jax.experimental.pallas import tpu as pltpu
from jax.experimental.pallas import tpu_sc as plsc

B, L, D = 4096, 50, 128
NC, NS = 2, 16
NW = NC * NS
PER_W = B // NW
NBUF = 8
NGROUP = PER_W // NBUF


@functools.partial(
    pl.kernel,
    mesh=plsc.VectorSubcoreMesh(core_axis_name="c", subcore_axis_name="s"),
    out_type=jax.ShapeDtypeStruct((B, L, D), jnp.float32),
    scratch_types=[
        pltpu.VMEM((PER_W, L), jnp.int32),
        pltpu.VMEM((2 * NBUF, L, D), jnp.float32),
    ]
    + [pltpu.SemaphoreType.DMA] * (2 * NBUF),
)
def _gather_kernel(idx_hbm, table_hbm, out_hbm, idx_v, bufs, *sems):
    gsems = sems[:NBUF]
    ssems = sems[NBUF:]
    wid = lax.axis_index("s") * NC + lax.axis_index("c")
    base = wid * PER_W
    pltpu.sync_copy(idx_hbm.at[wid], idx_v)

    for b in range(NBUF):
        pltpu.async_copy(table_hbm.at[idx_v.at[b]], bufs.at[b], gsems[b])

    def body(g, carry):
        p = lax.rem(g, 2)
        pn = 1 - p
        for b in range(NBUF):
            j = g * NBUF + b
            cur = p * NBUF + b
            nxt = pn * NBUF + b
            pltpu.make_async_copy(
                table_hbm.at[idx_v.at[j]], bufs.at[cur], gsems[b]
            ).wait()

            @pl.when(g > 0)
            def _drain():
                pltpu.make_async_copy(
                    bufs.at[nxt], out_hbm.at[base + j], ssems[b]
                ).wait()

            pltpu.async_copy(bufs.at[cur], out_hbm.at[base + j], ssems[b])

            @pl.when(g < NGROUP - 1)
            def _next_gather():
                pltpu.async_copy(
                    table_hbm.at[idx_v.at[j + NBUF]], bufs.at[nxt], gsems[b]
                )

        return carry

    lax.fori_loop(0, NGROUP, body, 0)

    for b in range(NBUF):
        pltpu.make_async_copy(
            bufs.at[b], out_hbm.at[base + PER_W - NBUF + b], ssems[b]
        ).wait()


def kernel(token_ids, weight):
    idx = token_ids.astype(jnp.int32).reshape(NW, PER_W, L)
    return _gather_kernel(idx, weight)


# speedup vs baseline: 10.6371x; 1.7720x over previous
import functools

import jax
import jax.numpy as jnp
from jax import lax
from jax.experimental import pallas as pl
from jax.experimental.pallas import tpu as pltpu
from jax.experimental.pallas import tpu_sc as plsc

B, L, D = 4096, 50, 128
NC, NS = 2, 16
NW = NC * NS
PER_W = B // NW
NBUF = 2
NGROUP = L // NBUF


@functools.partial(
    pl.kernel,
    mesh=plsc.VectorSubcoreMesh(core_axis_name="c", subcore_axis_name="s"),
    out_type=jax.ShapeDtypeStruct((L, B, D), jnp.float32),
    scratch_types=[
        pltpu.VMEM((L, PER_W), jnp.int32),
        pltpu.VMEM((2 * NBUF, PER_W, D), jnp.float32),
    ]
    + [pltpu.SemaphoreType.DMA] * (2 * NBUF),
)
def _gather_kernel(idx_hbm, table_hbm, out_hbm, idx_v, bufs, *sems):
    gsems = sems[:NBUF]
    ssems = sems[NBUF:]
    wid = lax.axis_index("s") * NC + lax.axis_index("c")
    base = wid * PER_W
    pltpu.sync_copy(idx_hbm.at[wid], idx_v)

    for b in range(NBUF):
        pltpu.async_copy(table_hbm.at[idx_v.at[b]], bufs.at[b], gsems[b])

    def body(g, carry):
        p = lax.rem(g, 2)
        pn = 1 - p
        for b in range(NBUF):
            j = g * NBUF + b
            cur = p * NBUF + b
            nxt = pn * NBUF + b
            pltpu.make_async_copy(
                table_hbm.at[idx_v.at[j]], bufs.at[cur], gsems[b]
            ).wait()

            @pl.when(g > 0)
            def _drain():
                pltpu.make_async_copy(
                    bufs.at[nxt], out_hbm.at[j].at[pl.ds(base, PER_W)], ssems[b]
                ).wait()

            pltpu.async_copy(
                bufs.at[cur], out_hbm.at[j].at[pl.ds(base, PER_W)], ssems[b]
            )

            @pl.when(g < NGROUP - 1)
            def _next_gather():
                pltpu.async_copy(
                    table_hbm.at[idx_v.at[j + NBUF]], bufs.at[nxt], gsems[b]
                )

        return carry

    lax.fori_loop(0, NGROUP, body, 0)

    for b in range(NBUF):
        pltpu.make_async_copy(
            bufs.at[b], out_hbm.at[0].at[pl.ds(base, PER_W)], ssems[b]
        ).wait()


def kernel(token_ids, weight):
    idx = token_ids.astype(jnp.int32).reshape(NW, PER_W, L).transpose(0, 2, 1)
    out = _gather_kernel(idx, weight)
    return out.transpose(1, 0, 2)
